# Initial kernel scaffold; baseline (speedup 1.0000x reference)
import jax
import jax.numpy as jnp
from jax.experimental import pallas as pl


def kernel(inp, W_nm1, b_nm1, W_nm2, b_nm2, Wg, bg, W1, b1, W2, b2):
    raise NotImplementedError("write your pallas kernel here")


# speedup vs baseline: 1.7806x; 1.7806x over previous
import functools

import jax
import jax.numpy as jnp
from jax import lax
from jax.experimental import pallas as pl
from jax.experimental.pallas import tpu as pltpu
from jax.experimental.pallas import tpu_sc as plsc

_T = 32768
_D = 8
_H = 16
_NW = 32
_CHUNK = _T // _NW
_GROUPS = _CHUNK // 16
_NPARAM = 624

_INV_SQRT2 = 0.7071067811865476


def _gelu_erf(u):
    a = jnp.abs(u)
    z = a * _INV_SQRT2
    t = 1.0 / (1.0 + 0.3275911 * z)
    poly = t * (0.254829592 + t * (-0.284496736 + t * (1.421413741
               + t * (-1.453152027 + t * 1.061405429))))
    erf = 1.0 - poly * jnp.exp(-z * z)
    return 0.5 * (u + a * erf)


def _sc_body(x_hbm, p_hbm, out_hbm, *refs):
    x_vs = refs[:_D]
    p_v, acc_v = refs[_D], refs[_D + 1]
    nc = plsc.get_sparse_core_info().num_cores
    wid = lax.axis_index("s") * nc + lax.axis_index("c")
    base = wid * _CHUNK
    for k in range(_D):
        pltpu.sync_copy(x_hbm.at[k, pl.ds(base, _CHUNK)], x_vs[k])
    pltpu.sync_copy(p_hbm, p_v)

    p = [p_v[i] for i in range(611)]
    w_nm1 = [[p[j * _D + k] for k in range(_D)] for j in range(_H)]
    b_nm1 = [p[128 + j] for j in range(_H)]
    w_nm2 = [[p[144 + i * _H + j] for j in range(_H)] for i in range(_D)]
    b_nm2 = [p[272 + i] for i in range(_D)]
    gvec = [p[280 + k] for k in range(_D)]
    gbias = p[288]
    w1 = [[[p[289 + e * 128 + j * _D + k] for k in range(_D)]
           for j in range(_H)] for e in range(2)]
    b1 = [[p[545 + e * _H + j] for j in range(_H)] for e in range(2)]
    vred = [[p[577 + e * _H + j] for j in range(_H)] for e in range(2)]
    cred = [p[609 + e] for e in range(2)]

    def group(gi, acc):
        tok = gi * 16
        xk = [x_vs[k][pl.ds(tok, 16)] for k in range(_D)]
        h = []
        for j in range(_H):
            a = xk[0] * w_nm1[j][0]
            for k in range(1, _D):
                a = a + xk[k] * w_nm1[j][k]
            h.append(jnp.maximum(a + b_nm1[j], 0.0))
        x2 = []
        for i in range(_D):
            a = h[0] * w_nm2[i][0]
            for j in range(1, _H):
                a = a + h[j] * w_nm2[i][j]
            x2.append(a + b_nm2[i])
        dlog = x2[0] * gvec[0]
        for k in range(1, _D):
            dlog = dlog + x2[k] * gvec[k]
        mask = (dlog + gbias) >= 0.0
        s = jnp.where(mask, cred[0], cred[1])
        for j in range(_H):
            a0 = x2[0] * w1[0][j][0]
            a1 = x2[0] * w1[1][j][0]
            for k in range(1, _D):
                a0 = a0 + x2[k] * w1[0][j][k]
                a1 = a1 + x2[k] * w1[1][j][k]
            pre = jnp.where(mask, a0 + b1[0][j], a1 + b1[1][j])
            hh = _gelu_erf(pre)
            s = s + hh * jnp.where(mask, vred[0][j], vred[1][j])
        return acc + s

    acc = lax.fori_loop(0, _GROUPS, group, jnp.zeros((16,), jnp.float32))
    acc_v[...] = acc
    pltpu.sync_copy(acc_v, out_hbm.at[wid])


@jax.jit
def kernel(inp, W_nm1, b_nm1, W_nm2, b_nm2, Wg, bg, W1, b1, W2, b2):
    g = Wg[0] - Wg[1]
    gb = bg[0] - bg[1]
    v = W2.sum(axis=1)
    c = b2.sum(axis=1)
    params = jnp.concatenate([
        W_nm1.ravel(), b_nm1, W_nm2.ravel(), b_nm2, g, gb[None],
        W1.ravel(), b1.ravel(), v.ravel(), c,
    ])
    params = jnp.pad(params, (0, _NPARAM - params.shape[0]))
    params = jnp.broadcast_to(params[:, None], (_NPARAM, 16))
    xt = inp.T

    run = pl.kernel(
        _sc_body,
        out_type=jax.ShapeDtypeStruct((_NW, 16), jnp.float32),
        mesh=plsc.VectorSubcoreMesh(core_axis_name="c", subcore_axis_name="s"),
        scratch_types=(
            [pltpu.VMEM((_CHUNK,), jnp.float32) for _ in range(_D)]
            + [pltpu.VMEM((_NPARAM, 16), jnp.float32),
               pltpu.VMEM((16,), jnp.float32)]
        ),
    )
    partials = run(xt, params)
    return jnp.sum(partials)
